# baseline (speedup 1.0000x reference)
import functools

import jax
import jax.numpy as jnp
from jax import lax
from jax.experimental import pallas as pl
from jax.experimental.pallas import tpu as pltpu
from jax.experimental.pallas import tpu_sc as plsc

N = 10000
E = 320000
F = 128
NG = 64
NCLS = 10
NLAYERS = 3
EPS = 1e-5

NSC = 2
NTILE = 16
NW = NSC * NTILE
EPW = E // NW
CH = 128
NCHUNK = 80
EPWP = NCHUNK * CH
EPAD = EPWP - EPW
NP = 10240
RPT = NP // NTILE


def _edge_body(x_hbm, src_hbm, dst_hbm, zeros_hbm, out_hbm,
               src_v, dst_v, rows_v, agg_sh, gsem):
    c = lax.axis_index("c")
    s = lax.axis_index("s")
    w = c * NTILE + s
    pltpu.sync_copy(zeros_hbm.at[pl.ds(s * RPT, RPT)],
                    agg_sh.at[pl.ds(s * RPT, RPT)])
    plsc.subcore_barrier()

    def body(j, carry):
        pltpu.sync_copy(src_hbm.at[w, j], src_v)
        pltpu.sync_copy(dst_hbm.at[w, j], dst_v)
        pltpu.async_copy(x_hbm.at[src_v], rows_v, gsem).wait()
        pltpu.sync_copy(rows_v, agg_sh.at[dst_v], add=True)
        return carry

    lax.fori_loop(0, NCHUNK, body, 0)
    plsc.subcore_barrier()
    pltpu.sync_copy(agg_sh.at[pl.ds(s * RPT, RPT)],
                    out_hbm.at[c, pl.ds(s * RPT, RPT)])


_edge_call = pl.kernel(
    _edge_body,
    out_type=jax.ShapeDtypeStruct((NSC, NP, F), jnp.float32),
    mesh=plsc.VectorSubcoreMesh(core_axis_name="c", subcore_axis_name="s"),
    scratch_types=[
        pltpu.VMEM((CH,), jnp.int32),
        pltpu.VMEM((CH,), jnp.int32),
        pltpu.VMEM((CH, F), jnp.float32),
        pltpu.VMEM_SHARED((NP, F), jnp.float32),
        pltpu.SemaphoreType.DMA,
    ],
)


def _embed_body(h_ref, we_ref, be_ref, o_ref):
    o_ref[...] = (jnp.dot(h_ref[...], we_ref[...],
                          preferred_element_type=jnp.float32) + be_ref[...])


_embed_call = pl.pallas_call(
    _embed_body,
    out_shape=jax.ShapeDtypeStruct((N, F), jnp.float32),
)


def _layer_body(x_ref, p_ref, w_ref, b_ref, g_ref, bt_ref, o_ref):
    agg = p_ref[0, :N, :] + p_ref[1, :N, :]
    z = 2.0 * x_ref[...] + agg
    y = jnp.dot(z, w_ref[...], preferred_element_type=jnp.float32) + b_ref[...]
    mean = jnp.mean(y, axis=0, keepdims=True)
    d = y - mean
    var = jnp.mean(d * d, axis=0, keepdims=True)
    yn = d * lax.rsqrt(var + EPS) * g_ref[...] + bt_ref[...]
    o_ref[...] = jnp.maximum(yn, 0.0)


_layer_call = pl.pallas_call(
    _layer_body,
    out_shape=jax.ShapeDtypeStruct((N, F), jnp.float32),
)


def _pool_body(x_ref, batch_ref, wfc_ref, bfc_ref, o_ref):
    gids = lax.broadcasted_iota(jnp.int32, (NG, N), 0)
    onehot = (gids == batch_ref[...]).astype(jnp.float32)
    pooled = jnp.dot(onehot, x_ref[...], preferred_element_type=jnp.float32)
    o_ref[...] = (jnp.dot(pooled, wfc_ref[...],
                          preferred_element_type=jnp.float32) + bfc_ref[...])


_pool_call = pl.pallas_call(
    _pool_body,
    out_shape=jax.ShapeDtypeStruct((NG, NCLS), jnp.float32),
)


def kernel(h, edge_index, pair_info, batch, W_emb, b_emb, W, b, gamma, beta,
           Wfc, bfc):
    srcw = pair_info[0].reshape(NW, EPW)
    dstw = pair_info[1].reshape(NW, EPW)
    pad_src = jnp.zeros((NW, EPAD), jnp.int32)
    pad_dst = jnp.broadcast_to(
        N + (jnp.arange(EPAD, dtype=jnp.int32) % (NP - N)), (NW, EPAD))
    src = jnp.concatenate([srcw, pad_src], axis=1).reshape(NW, NCHUNK, CH)
    dst = jnp.concatenate([dstw, pad_dst], axis=1).reshape(NW, NCHUNK, CH)
    zeros = jnp.zeros((NP, F), jnp.float32)
    x = _embed_call(h, W_emb, b_emb.reshape(1, F))
    for l in range(NLAYERS):
        parts = _edge_call(x, src, dst, zeros)
        x = _layer_call(x, parts, W[l], b[l].reshape(1, F),
                        gamma[l].reshape(1, F), beta[l].reshape(1, F))
    return _pool_call(x, batch.reshape(1, N), Wfc, bfc.reshape(1, NCLS))


# speedup vs baseline: 1.6421x; 1.6421x over previous
import functools

import jax
import jax.numpy as jnp
from jax import lax
from jax.experimental import pallas as pl
from jax.experimental.pallas import tpu as pltpu
from jax.experimental.pallas import tpu_sc as plsc

N = 10000
E = 320000
F = 128
NG = 64
NCLS = 10
NLAYERS = 3
EPS = 1e-5

NSC = 2
NTILE = 16
NW = NSC * NTILE
EPW = E // NW
CH = 80
NCHUNK = EPW // CH
NP = 10240
RPT = NP // NTILE


def _edge_body(x_hbm, src_hbm, dst_hbm, zeros_hbm, out_hbm,
               src_v, dst_v, rows_v, agg_sh, gsem):
    c = lax.axis_index("c")
    s = lax.axis_index("s")
    w = c * NTILE + s
    base = w * EPW
    pltpu.sync_copy(zeros_hbm.at[pl.ds(s * RPT, RPT)],
                    agg_sh.at[pl.ds(s * RPT, RPT)])
    plsc.subcore_barrier()

    def body(j, carry):
        off = base + j * CH
        pltpu.sync_copy(src_hbm.at[pl.ds(off, CH)], src_v)
        pltpu.sync_copy(dst_hbm.at[pl.ds(off, CH)], dst_v)
        pltpu.async_copy(x_hbm.at[src_v], rows_v, gsem).wait()
        pltpu.sync_copy(rows_v, agg_sh.at[dst_v], add=True)
        return carry

    lax.fori_loop(0, NCHUNK, body, 0)
    plsc.subcore_barrier()
    pltpu.sync_copy(agg_sh.at[pl.ds(s * RPT, RPT)],
                    out_hbm.at[c, pl.ds(s * RPT, RPT)])


_edge_call = pl.kernel(
    _edge_body,
    out_type=jax.ShapeDtypeStruct((NSC, NP, F), jnp.float32),
    mesh=plsc.VectorSubcoreMesh(core_axis_name="c", subcore_axis_name="s"),
    scratch_types=[
        pltpu.VMEM((CH,), jnp.int32),
        pltpu.VMEM((CH,), jnp.int32),
        pltpu.VMEM((CH, F), jnp.float32),
        pltpu.VMEM_SHARED((NP, F), jnp.float32),
        pltpu.SemaphoreType.DMA,
    ],
)


def _embed_body(h_ref, we_ref, be_ref, o_ref):
    o_ref[...] = (jnp.dot(h_ref[...], we_ref[...],
                          preferred_element_type=jnp.float32) + be_ref[...])


_embed_call = pl.pallas_call(
    _embed_body,
    out_shape=jax.ShapeDtypeStruct((N, F), jnp.float32),
)


def _layer_body(x_ref, p_ref, w_ref, b_ref, g_ref, bt_ref, o_ref):
    agg = p_ref[0, :N, :] + p_ref[1, :N, :]
    z = 2.0 * x_ref[...] + agg
    y = jnp.dot(z, w_ref[...], preferred_element_type=jnp.float32) + b_ref[...]
    mean = jnp.mean(y, axis=0, keepdims=True)
    d = y - mean
    var = jnp.mean(d * d, axis=0, keepdims=True)
    yn = d * lax.rsqrt(var + EPS) * g_ref[...] + bt_ref[...]
    o_ref[...] = jnp.maximum(yn, 0.0)


_layer_call = pl.pallas_call(
    _layer_body,
    out_shape=jax.ShapeDtypeStruct((N, F), jnp.float32),
)


def _pool_body(x_ref, batch_ref, wfc_ref, bfc_ref, o_ref):
    gids = lax.broadcasted_iota(jnp.int32, (NG, N), 0)
    onehot = (gids == batch_ref[...]).astype(jnp.float32)
    pooled = jnp.dot(onehot, x_ref[...], preferred_element_type=jnp.float32)
    o_ref[...] = (jnp.dot(pooled, wfc_ref[...],
                          preferred_element_type=jnp.float32) + bfc_ref[...])


_pool_call = pl.pallas_call(
    _pool_body,
    out_shape=jax.ShapeDtypeStruct((NG, NCLS), jnp.float32),
)


def kernel(h, edge_index, pair_info, batch, W_emb, b_emb, W, b, gamma, beta,
           Wfc, bfc):
    src = pair_info[0]
    dst = pair_info[1]
    zeros = jnp.zeros((NP, F), jnp.float32)
    x = _embed_call(h, W_emb, b_emb.reshape(1, F))
    for l in range(NLAYERS):
        parts = _edge_call(x, src, dst, zeros)
        x = _layer_call(x, parts, W[l], b[l].reshape(1, F),
                        gamma[l].reshape(1, F), beta[l].reshape(1, F))
    return _pool_call(x, batch.reshape(1, N), Wfc, bfc.reshape(1, NCLS))
